# Initial kernel scaffold; baseline (speedup 1.0000x reference)
import functools

import jax
import jax.numpy as jnp
from jax import lax
from jax.experimental import pallas as pl
from jax.experimental.pallas import tpu as pltpu
from jax.experimental.pallas import tpu_sc as plsc

N = 10000
E = 320000
DN = 128
DE = 16

NC, NS = 2, 16
NW = NC * NS
CH = 80

EPW = E // NW
NCHUNK = EPW // CH


def _gather_body(x_hbm, src_hbm, dst_hbm, hs_hbm, hd_hbm, idx_v, rows_v, sem):
    c = lax.axis_index("c")
    s = lax.axis_index("s")
    wid = s * NC + c
    base = wid * EPW

    def body(i, carry):
        off = base + i * CH
        pltpu.sync_copy(src_hbm.at[pl.ds(off, CH)], idx_v)
        pltpu.async_copy(x_hbm.at[idx_v], rows_v, sem).wait()
        pltpu.sync_copy(rows_v, hs_hbm.at[pl.ds(off, CH)])
        pltpu.sync_copy(dst_hbm.at[pl.ds(off, CH)], idx_v)
        pltpu.async_copy(x_hbm.at[idx_v], rows_v, sem).wait()
        pltpu.sync_copy(rows_v, hd_hbm.at[pl.ds(off, CH)])
        return carry

    lax.fori_loop(0, NCHUNK, body, 0)


def _sc_gather(x, src, dst):
    mesh = plsc.VectorSubcoreMesh(core_axis_name="c", subcore_axis_name="s")
    f = pl.kernel(
        _gather_body,
        out_type=(
            jax.ShapeDtypeStruct((E, DN), jnp.float32),
            jax.ShapeDtypeStruct((E, DN), jnp.float32),
        ),
        mesh=mesh,
        scratch_types=[
            pltpu.VMEM((CH,), jnp.int32),
            pltpu.VMEM((CH, DN), jnp.float32),
            pltpu.SemaphoreType.DMA,
        ],
    )
    return f(x, src, dst)


BE = 2560


def _edge_body(hs_ref, hd_ref, ea_ref, wn1_ref, bn1_ref, wn2_ref, we1_ref,
               be1_ref, we2_ref, wc_ref, wue_ref, m_ref):
    hs = hs_ref[...]
    hd = hd_ref[...]
    m1 = jnp.maximum(hs @ wn1_ref[...] + bn1_ref[...], 0.0) @ wn2_ref[...]
    u = (hs * hd) @ wue_ref[...]
    e_h = 0.8 * ea_ref[...] + 0.2 * u
    t = jnp.maximum(e_h @ we1_ref[...] + be1_ref[...], 0.0)
    m2 = t @ we2_ref[...]
    m_ref[...] = jnp.tanh((m1 * m2) @ wc_ref[...])


def _tc_edge(hs, hd, ea, wn1, bn1, wn2, we1, be1, we2, wc, wue):
    full = lambda shape: pl.BlockSpec(shape, lambda i: (0,) * len(shape))
    return pl.pallas_call(
        _edge_body,
        grid=(E // BE,),
        in_specs=[
            pl.BlockSpec((BE, DN), lambda i: (i, 0)),
            pl.BlockSpec((BE, DN), lambda i: (i, 0)),
            pl.BlockSpec((BE, DE), lambda i: (i, 0)),
            full((DN, DN)), full((1, DN)), full((DN, DN)),
            full((DE, DN)), full((1, DN)), full((DN, DN)),
            full((DN, DN)), full((DN, DE)),
        ],
        out_specs=pl.BlockSpec((BE, DN), lambda i: (i, 0)),
        out_shape=jax.ShapeDtypeStruct((E, DN), jnp.float32),
    )(hs, hd, ea, wn1, bn1, wn2, we1, be1, we2, wc, wue)


EPH = E // NC
EPW2 = EPH // NS
NCHUNK2 = EPW2 // CH
RPT = N // NS


def _scatter_body(m_hbm, dst_hbm, zero_hbm, out_hbm, idx_v, rows_v, acc, sem):
    c = lax.axis_index("c")
    s = lax.axis_index("s")
    pltpu.sync_copy(zero_hbm.at[pl.ds(s * RPT, RPT)], acc.at[pl.ds(s * RPT, RPT)])
    plsc.subcore_barrier()

    base = c * EPH + s * EPW2

    def body(i, carry):
        off = base + i * CH
        pltpu.sync_copy(dst_hbm.at[pl.ds(off, CH)], idx_v)
        pltpu.sync_copy(m_hbm.at[pl.ds(off, CH)], rows_v)
        pltpu.sync_copy(rows_v, acc.at[idx_v], add=True)
        return carry

    lax.fori_loop(0, NCHUNK2, body, 0)
    plsc.subcore_barrier()
    pltpu.sync_copy(acc.at[pl.ds(s * RPT, RPT)], out_hbm.at[c, pl.ds(s * RPT, RPT)])


def _sc_scatter(m, dst, zeros_n):
    mesh = plsc.VectorSubcoreMesh(core_axis_name="c", subcore_axis_name="s")
    f = pl.kernel(
        _scatter_body,
        out_type=jax.ShapeDtypeStruct((NC, N, DN), jnp.float32),
        mesh=mesh,
        scratch_types=[
            pltpu.VMEM((CH,), jnp.int32),
            pltpu.VMEM((CH, DN), jnp.float32),
            pltpu.VMEM_SHARED((N, DN), jnp.float32),
            pltpu.SemaphoreType.DMA,
        ],
    )
    return f(m, dst, zeros_n)


BN = 2000


def _combine_body(p_ref, x_ref, o_ref):
    o_ref[...] = p_ref[0] + p_ref[1] + x_ref[...]


def _tc_combine(p, x):
    return pl.pallas_call(
        _combine_body,
        grid=(N // BN,),
        in_specs=[
            pl.BlockSpec((NC, BN, DN), lambda i: (0, i, 0)),
            pl.BlockSpec((BN, DN), lambda i: (i, 0)),
        ],
        out_specs=pl.BlockSpec((BN, DN), lambda i: (i, 0)),
        out_shape=jax.ShapeDtypeStruct((N, DN), jnp.float32),
    )(p, x)


def kernel(x, edge_index, edge_attr, W_node1, b_node1, W_node2, W_edge1,
           b_edge1, W_edge2, W_combine, W_update_edge):
    ei = edge_index.astype(jnp.int32)
    src = ei[0]
    dst = ei[1]
    hs, hd = _sc_gather(x, src, dst)
    m = _tc_edge(hs, hd, edge_attr,
                 W_node1, b_node1.reshape(1, DN), W_node2,
                 W_edge1, b_edge1.reshape(1, DN), W_edge2,
                 W_combine, W_update_edge)
    p = _sc_scatter(m, dst, jnp.zeros((N, DN), jnp.float32))
    return _tc_combine(p, x)


# speedup vs baseline: 2.6519x; 2.6519x over previous
import functools

import jax
import jax.numpy as jnp
from jax import lax
from jax.experimental import pallas as pl
from jax.experimental.pallas import tpu as pltpu
from jax.experimental.pallas import tpu_sc as plsc

N = 10000
E = 320000
DN = 128
DE = 16

NC, NS = 2, 16
NW = NC * NS
CH = 80

EPW = E // NW
NCHUNK = EPW // CH


def _gather_body(x_hbm, src_hbm, dst_hbm, hs_hbm, hd_hbm, idx_v, rows_v, sem):
    c = lax.axis_index("c")
    s = lax.axis_index("s")
    wid = s * NC + c
    base = wid * EPW

    def body(i, carry):
        off = base + i * CH
        pltpu.sync_copy(src_hbm.at[pl.ds(off, CH)], idx_v)
        pltpu.async_copy(x_hbm.at[idx_v], rows_v, sem).wait()
        pltpu.sync_copy(rows_v, hs_hbm.at[pl.ds(off, CH)])
        pltpu.sync_copy(dst_hbm.at[pl.ds(off, CH)], idx_v)
        pltpu.async_copy(x_hbm.at[idx_v], rows_v, sem).wait()
        pltpu.sync_copy(rows_v, hd_hbm.at[pl.ds(off, CH)])
        return carry

    lax.fori_loop(0, NCHUNK, body, 0)


def _sc_gather(x, src, dst):
    mesh = plsc.VectorSubcoreMesh(core_axis_name="c", subcore_axis_name="s")
    f = pl.kernel(
        _gather_body,
        out_type=(
            jax.ShapeDtypeStruct((E, DN), jnp.float32),
            jax.ShapeDtypeStruct((E, DN), jnp.float32),
        ),
        mesh=mesh,
        scratch_types=[
            pltpu.VMEM((CH,), jnp.int32),
            pltpu.VMEM((CH, DN), jnp.float32),
            pltpu.SemaphoreType.DMA,
        ],
    )
    return f(x, src, dst)


BE = 2560


def _edge_body(hs_ref, hd_ref, ea_ref, wn1_ref, bn1_ref, wn2_ref, we1_ref,
               be1_ref, we2_ref, wc_ref, wue_ref, m_ref):
    hs = hs_ref[...]
    hd = hd_ref[...]
    m1 = jnp.maximum(hs @ wn1_ref[...] + bn1_ref[...], 0.0) @ wn2_ref[...]
    u = (hs * hd) @ wue_ref[...]
    e_h = 0.8 * ea_ref[...] + 0.2 * u
    t = jnp.maximum(e_h @ we1_ref[...] + be1_ref[...], 0.0)
    m2 = t @ we2_ref[...]
    m_ref[...] = jnp.tanh((m1 * m2) @ wc_ref[...])


def _tc_edge(hs, hd, ea, wn1, bn1, wn2, we1, be1, we2, wc, wue):
    full = lambda shape: pl.BlockSpec(shape, lambda i: (0,) * len(shape))
    return pl.pallas_call(
        _edge_body,
        grid=(E // BE,),
        in_specs=[
            pl.BlockSpec((BE, DN), lambda i: (i, 0)),
            pl.BlockSpec((BE, DN), lambda i: (i, 0)),
            pl.BlockSpec((BE, DE), lambda i: (i, 0)),
            full((DN, DN)), full((1, DN)), full((DN, DN)),
            full((DE, DN)), full((1, DN)), full((DN, DN)),
            full((DN, DN)), full((DN, DE)),
        ],
        out_specs=pl.BlockSpec((BE, DN), lambda i: (i, 0)),
        out_shape=jax.ShapeDtypeStruct((E, DN), jnp.float32),
    )(hs, hd, ea, wn1, bn1, wn2, we1, be1, we2, wc, wue)


EPH = E // NC
EPW2 = EPH // NS
NCHUNK2 = EPW2 // CH
N_PAD = 10240
RPT = N_PAD // NS


def _scatter_body(m_hbm, dst_hbm, zero_hbm, out_hbm, idx_v, rows_v, acc, sem):
    c = lax.axis_index("c")
    s = lax.axis_index("s")
    pltpu.sync_copy(zero_hbm.at[pl.ds(s * RPT, RPT)], acc.at[pl.ds(s * RPT, RPT)])
    plsc.subcore_barrier()

    base = c * EPH + s * EPW2

    def body(i, carry):
        off = base + i * CH
        pltpu.sync_copy(dst_hbm.at[pl.ds(off, CH)], idx_v)
        pltpu.sync_copy(m_hbm.at[pl.ds(off, CH)], rows_v)
        pltpu.sync_copy(rows_v, acc.at[idx_v], add=True)
        return carry

    lax.fori_loop(0, NCHUNK2, body, 0)
    plsc.subcore_barrier()
    pltpu.sync_copy(acc.at[pl.ds(s * RPT, RPT)], out_hbm.at[c, pl.ds(s * RPT, RPT)])


def _sc_scatter(m, dst, zeros_n):
    mesh = plsc.VectorSubcoreMesh(core_axis_name="c", subcore_axis_name="s")
    f = pl.kernel(
        _scatter_body,
        out_type=jax.ShapeDtypeStruct((NC, N_PAD, DN), jnp.float32),
        mesh=mesh,
        scratch_types=[
            pltpu.VMEM((CH,), jnp.int32),
            pltpu.VMEM((CH, DN), jnp.float32),
            pltpu.VMEM_SHARED((N_PAD, DN), jnp.float32),
            pltpu.SemaphoreType.DMA,
        ],
    )
    return f(m, dst, zeros_n)


BN = 2000


def _combine_body(p_ref, x_ref, o_ref):
    o_ref[...] = p_ref[0] + p_ref[1] + x_ref[...]


def _tc_combine(p, x):
    return pl.pallas_call(
        _combine_body,
        grid=(N // BN,),
        in_specs=[
            pl.BlockSpec((NC, BN, DN), lambda i: (0, i, 0)),
            pl.BlockSpec((BN, DN), lambda i: (i, 0)),
        ],
        out_specs=pl.BlockSpec((BN, DN), lambda i: (i, 0)),
        out_shape=jax.ShapeDtypeStruct((N, DN), jnp.float32),
    )(p, x)


def kernel(x, edge_index, edge_attr, W_node1, b_node1, W_node2, W_edge1,
           b_edge1, W_edge2, W_combine, W_update_edge):
    ei = edge_index.astype(jnp.int32)
    src = ei[0]
    dst = ei[1]
    hs, hd = _sc_gather(x, src, dst)
    m = _tc_edge(hs, hd, edge_attr,
                 W_node1, b_node1.reshape(1, DN), W_node2,
                 W_edge1, b_edge1.reshape(1, DN), W_edge2,
                 W_combine, W_update_edge)
    p = _sc_scatter(m, dst, jnp.zeros((N_PAD, DN), jnp.float32))
    return _tc_combine(p, x)


# speedup vs baseline: 2.9234x; 1.1023x over previous
import jax
import jax.numpy as jnp
from jax import lax
from jax.experimental import pallas as pl
from jax.experimental.pallas import tpu as pltpu
from jax.experimental.pallas import tpu_sc as plsc

N = 10000
E = 320000
DN = 128
DE = 16
DP = DN // 2

NC, NS = 2, 16
NW = NC * NS
CH = 80
EPW = E // NW
NCHUNK = EPW // CH
N_PAD = 10240
TR = N_PAD // NS


def _gather_body(x_hbm, src_hbm, dst_hbm, hs_hbm, hd_hbm,
                 idx_s, idx_d, rs0, rd0, sem_g0, sem_g1):
    c = lax.axis_index("c")
    s = lax.axis_index("s")
    w = c * NS + s
    base = w * EPW
    pltpu.sync_copy(src_hbm.at[w], idx_s)
    pltpu.sync_copy(dst_hbm.at[w], idx_d)

    def body(i, carry):
        off = base + i * CH
        pltpu.async_copy(x_hbm.at[idx_s.at[i]], rs0, sem_g0).wait()
        pltpu.sync_copy(rs0, hs_hbm.at[pl.ds(off, CH)])
        pltpu.async_copy(x_hbm.at[idx_d.at[i]], rd0, sem_g1).wait()
        pltpu.sync_copy(rd0, hd_hbm.at[pl.ds(off, CH)])
        return carry

    lax.fori_loop(0, NCHUNK, body, 0)


def _sc_gather(x, src3, dst3):
    mesh = plsc.VectorSubcoreMesh(core_axis_name="c", subcore_axis_name="s")
    f = pl.kernel(
        _gather_body,
        out_type=(
            jax.ShapeDtypeStruct((E, DN), jnp.float32),
            jax.ShapeDtypeStruct((E, DN), jnp.float32),
        ),
        mesh=mesh,
        scratch_types=[
            pltpu.VMEM((NCHUNK, CH), jnp.int32),
            pltpu.VMEM((NCHUNK, CH), jnp.int32),
            pltpu.VMEM((CH, DN), jnp.float32),
            pltpu.VMEM((CH, DN), jnp.float32),
            pltpu.SemaphoreType.DMA,
            pltpu.SemaphoreType.DMA,
        ],
    )
    return f(x, src3, dst3)


BE = 2560


def _edge_body(hs_ref, hd_ref, ea_ref, wn1_ref, bn1_ref, wn2_ref, we1_ref,
               be1_ref, we2_ref, wc_ref, wue_ref, m_ref):
    hs = hs_ref[...]
    hd = hd_ref[...]
    m1 = jnp.maximum(hs @ wn1_ref[...] + bn1_ref[...], 0.0) @ wn2_ref[...]
    u = (hs * hd) @ wue_ref[...]
    e_h = 0.8 * ea_ref[...] + 0.2 * u
    t = jnp.maximum(e_h @ we1_ref[...] + be1_ref[...], 0.0)
    m2 = t @ we2_ref[...]
    m_ref[...] = jnp.tanh((m1 * m2) @ wc_ref[...])


def _tc_edge(hs, hd, ea, wn1, bn1, wn2, we1, be1, we2, wc, wue):
    full = lambda shape: pl.BlockSpec(shape, lambda i: (0,) * len(shape))
    return pl.pallas_call(
        _edge_body,
        grid=(E // BE,),
        in_specs=[
            pl.BlockSpec((BE, DN), lambda i: (i, 0)),
            pl.BlockSpec((BE, DN), lambda i: (i, 0)),
            pl.BlockSpec((BE, DE), lambda i: (i, 0)),
            full((DN, DN)), full((1, DN)), full((DN, DN)),
            full((DE, DN)), full((1, DN)), full((DN, DN)),
            full((DN, DN)), full((DN, DE)),
        ],
        out_specs=pl.BlockSpec((BE, DN), lambda i: (i, 0)),
        out_shape=jax.ShapeDtypeStruct((E, DN), jnp.float32),
    )(hs, hd, ea, wn1, bn1, wn2, we1, be1, we2, wc, wue)


def _scatter_body(m_hbm, dst_hbm, zero_hbm, out_hbm, idx, mb0, mb1, acc,
                  sem_l0, sem_l1):
    c = lax.axis_index("c")
    s = lax.axis_index("s")
    w = c * NS + s
    base = w * EPW
    pltpu.sync_copy(zero_hbm.at[pl.ds(s * TR, TR)], acc.at[pl.ds(s * TR, TR)])
    plsc.subcore_barrier()

    def body(i, carry):
        pltpu.sync_copy(dst_hbm.at[w, i], idx)
        pltpu.sync_copy(m_hbm.at[pl.ds(base + i * CH, CH)], mb0)
        pltpu.sync_copy(mb0, acc.at[idx], add=True)
        return carry

    lax.fori_loop(0, NCHUNK, body, 0)
    del mb1, sem_l0, sem_l1

    plsc.subcore_barrier()
    pltpu.sync_copy(acc.at[pl.ds(s * TR, TR)], out_hbm.at[c, pl.ds(s * TR, TR)])


def _sc_scatter(m, dst3, zeros_n):
    mesh = plsc.VectorSubcoreMesh(core_axis_name="c", subcore_axis_name="s")
    f = pl.kernel(
        _scatter_body,
        out_type=jax.ShapeDtypeStruct((NC, N_PAD, DN), jnp.float32),
        mesh=mesh,
        scratch_types=[
            pltpu.VMEM((CH,), jnp.int32),
            pltpu.VMEM((CH, DN), jnp.float32),
            pltpu.VMEM((CH, DN), jnp.float32),
            pltpu.VMEM_SHARED((N_PAD, DN), jnp.float32),
            pltpu.SemaphoreType.DMA,
            pltpu.SemaphoreType.DMA,
        ],
    )
    return f(m, dst3, zeros_n)


BN = 2000


def _combine_body(p_ref, x_ref, o_ref):
    o_ref[...] = p_ref[0] + p_ref[1] + x_ref[...]


def _tc_combine(p, x):
    return pl.pallas_call(
        _combine_body,
        grid=(N // BN,),
        in_specs=[
            pl.BlockSpec((NC, BN, DN), lambda i: (0, i, 0)),
            pl.BlockSpec((BN, DN), lambda i: (i, 0)),
        ],
        out_specs=pl.BlockSpec((BN, DN), lambda i: (i, 0)),
        out_shape=jax.ShapeDtypeStruct((N, DN), jnp.float32),
    )(p, x)


def kernel(x, edge_index, edge_attr, W_node1, b_node1, W_node2, W_edge1,
           b_edge1, W_edge2, W_combine, W_update_edge):
    ei = edge_index.astype(jnp.int32)
    src3 = ei[0].reshape(NW, NCHUNK, CH)
    dst3 = ei[1].reshape(NW, NCHUNK, CH)
    hs, hd = _sc_gather(x, src3, dst3)
    m = _tc_edge(hs, hd, edge_attr,
                 W_node1, b_node1.reshape(1, DN), W_node2,
                 W_edge1, b_edge1.reshape(1, DN), W_edge2,
                 W_combine, W_update_edge)
    p = _sc_scatter(m, dst3, jnp.zeros((N_PAD, DN), jnp.float32))
    return _tc_combine(p, x)


# speedup vs baseline: 4.0922x; 1.3998x over previous
import jax
import jax.numpy as jnp
from jax import lax
from jax.experimental import pallas as pl
from jax.experimental.pallas import tpu as pltpu
from jax.experimental.pallas import tpu_sc as plsc

N = 10000
E = 320000
DN = 128
DE = 16
DP = DN // 2

NC, NS = 2, 16
NW = NC * NS
CH = 80
EPW = E // NW
NCHUNK = EPW // CH
N_PAD = 10240
TR = N_PAD // NS


def _gather_body(x_hbm, src_hbm, dst_hbm, hs_hbm, hd_hbm,
                 idx_s, idx_d, rs0, rs1, rd0, rd1,
                 sgs0, sgs1, sgd0, sgd1, sss0, sss1, ssd0, ssd1):
    c = lax.axis_index("c")
    s = lax.axis_index("s")
    w = c * NS + s
    base = w * EPW
    pltpu.sync_copy(src_hbm.at[w], idx_s)
    pltpu.sync_copy(dst_hbm.at[w], idx_d)

    rs = (rs0, rs1)
    rd = (rd0, rd1)
    sgs = (sgs0, sgs1)
    sgd = (sgd0, sgd1)
    sss = (sss0, sss1)
    ssd = (ssd0, ssd1)

    def issue_g(i, b):
        pltpu.async_copy(x_hbm.at[idx_s.at[i]], rs[b], sgs[b])
        pltpu.async_copy(x_hbm.at[idx_d.at[i]], rd[b], sgd[b])

    def wait_g(b):
        pltpu.make_async_copy(x_hbm.at[idx_s.at[0]], rs[b], sgs[b]).wait()
        pltpu.make_async_copy(x_hbm.at[idx_d.at[0]], rd[b], sgd[b]).wait()

    def issue_s(i, b):
        off = base + i * CH
        pltpu.async_copy(rs[b], hs_hbm.at[pl.ds(off, CH)], sss[b])
        pltpu.async_copy(rd[b], hd_hbm.at[pl.ds(off, CH)], ssd[b])

    def wait_s(b):
        pltpu.make_async_copy(rs[b], hs_hbm.at[pl.ds(base, CH)], sss[b]).wait()
        pltpu.make_async_copy(rd[b], hd_hbm.at[pl.ds(base, CH)], ssd[b]).wait()

    issue_g(0, 0)
    wait_g(0)
    issue_g(1, 1)
    issue_s(0, 0)
    wait_g(1)
    wait_s(0)
    issue_g(2, 0)
    issue_s(1, 1)

    def body(j, carry):
        for b in (0, 1):
            i = 2 * j + b
            wait_g(b)
            wait_s(1 - b)
            issue_g(i + 1, 1 - b)
            issue_s(i, b)
        return carry

    lax.fori_loop(1, (NCHUNK - 1) // 2, body, 0)
    wait_g(0)
    wait_s(1)
    issue_s(NCHUNK - 1, 0)
    wait_s(0)


def _sc_gather(x, src3, dst3):
    mesh = plsc.VectorSubcoreMesh(core_axis_name="c", subcore_axis_name="s")
    f = pl.kernel(
        _gather_body,
        out_type=(
            jax.ShapeDtypeStruct((E, DN), jnp.float32),
            jax.ShapeDtypeStruct((E, DN), jnp.float32),
        ),
        mesh=mesh,
        scratch_types=[
            pltpu.VMEM((NCHUNK, CH), jnp.int32),
            pltpu.VMEM((NCHUNK, CH), jnp.int32),
            pltpu.VMEM((CH, DN), jnp.float32),
            pltpu.VMEM((CH, DN), jnp.float32),
            pltpu.VMEM((CH, DN), jnp.float32),
            pltpu.VMEM((CH, DN), jnp.float32),
        ] + [pltpu.SemaphoreType.DMA] * 8,
    )
    return f(x, src3, dst3)


BE = 2560


def _edge_body(hs_ref, hd_ref, ea_ref, wn1_ref, bn1_ref, wn2_ref, we1_ref,
               be1_ref, we2_ref, wc_ref, wue_ref, m_ref):
    hs = hs_ref[...]
    hd = hd_ref[...]
    m1 = jnp.maximum(hs @ wn1_ref[...] + bn1_ref[...], 0.0) @ wn2_ref[...]
    u = (hs * hd) @ wue_ref[...]
    e_h = 0.8 * ea_ref[...] + 0.2 * u
    t = jnp.maximum(e_h @ we1_ref[...] + be1_ref[...], 0.0)
    m2 = t @ we2_ref[...]
    m_ref[...] = jnp.tanh((m1 * m2) @ wc_ref[...])


def _tc_edge(hs, hd, ea, wn1, bn1, wn2, we1, be1, we2, wc, wue):
    full = lambda shape: pl.BlockSpec(shape, lambda i: (0,) * len(shape))
    return pl.pallas_call(
        _edge_body,
        grid=(E // BE,),
        in_specs=[
            pl.BlockSpec((BE, DN), lambda i: (i, 0)),
            pl.BlockSpec((BE, DN), lambda i: (i, 0)),
            pl.BlockSpec((BE, DE), lambda i: (i, 0)),
            full((DN, DN)), full((1, DN)), full((DN, DN)),
            full((DE, DN)), full((1, DN)), full((DN, DN)),
            full((DN, DN)), full((DN, DE)),
        ],
        out_specs=pl.BlockSpec((BE, DN), lambda i: (i, 0)),
        out_shape=jax.ShapeDtypeStruct((E, DN), jnp.float32),
    )(hs, hd, ea, wn1, bn1, wn2, we1, be1, we2, wc, wue)


def _scatter_body(m_hbm, dst_hbm, zero_hbm, out_hbm, idx_all, idx0, idx1,
                  mb0, mb1, acc, sl0, sl1):
    c = lax.axis_index("c")
    s = lax.axis_index("s")
    w = c * NS + s
    base = w * EPW
    pltpu.sync_copy(zero_hbm.at[pl.ds(s * TR, TR)], acc.at[pl.ds(s * TR, TR)])
    pltpu.sync_copy(dst_hbm.at[w], idx_all)

    idx = (idx0, idx1)
    mb = (mb0, mb1)
    sl = (sl0, sl1)

    def issue_l(i, b):
        pltpu.async_copy(m_hbm.at[pl.ds(base + i * CH, CH)], mb[b], sl[b])

    def wait_l(b):
        pltpu.make_async_copy(m_hbm.at[pl.ds(base, CH)], mb[b], sl[b]).wait()

    def scatter_add(i, b):
        for k in range(CH // 16):
            idx[b][pl.ds(k * 16, 16)] = idx_all[i, pl.ds(k * 16, 16)]
        pltpu.sync_copy(mb[b], acc.at[idx[b]], add=True)

    issue_l(0, 0)
    issue_l(1, 1)
    plsc.subcore_barrier()

    def body(j, carry):
        for b in (0, 1):
            i = 2 * j + b
            wait_l(b)
            scatter_add(i, b)
            issue_l(i + 2, b)
        return carry

    lax.fori_loop(0, (NCHUNK - 3) // 2, body, 0)
    wait_l(0)
    scatter_add(NCHUNK - 3, 0)
    issue_l(NCHUNK - 1, 0)
    wait_l(1)
    scatter_add(NCHUNK - 2, 1)
    wait_l(0)
    scatter_add(NCHUNK - 1, 0)

    plsc.subcore_barrier()
    pltpu.sync_copy(acc.at[pl.ds(s * TR, TR)], out_hbm.at[c, pl.ds(s * TR, TR)])


def _sc_scatter(m, dst3, zeros_n):
    mesh = plsc.VectorSubcoreMesh(core_axis_name="c", subcore_axis_name="s")
    f = pl.kernel(
        _scatter_body,
        out_type=jax.ShapeDtypeStruct((NC, N_PAD, DN), jnp.float32),
        mesh=mesh,
        scratch_types=[
            pltpu.VMEM((NCHUNK, CH), jnp.int32),
            pltpu.VMEM((CH,), jnp.int32),
            pltpu.VMEM((CH,), jnp.int32),
            pltpu.VMEM((CH, DN), jnp.float32),
            pltpu.VMEM((CH, DN), jnp.float32),
            pltpu.VMEM_SHARED((N_PAD, DN), jnp.float32),
        ] + [pltpu.SemaphoreType.DMA] * 2,
    )
    return f(m, dst3, zeros_n)


BN = 2000


def _combine_body(p_ref, x_ref, o_ref):
    o_ref[...] = p_ref[0] + p_ref[1] + x_ref[...]


def _tc_combine(p, x):
    return pl.pallas_call(
        _combine_body,
        grid=(N // BN,),
        in_specs=[
            pl.BlockSpec((NC, BN, DN), lambda i: (0, i, 0)),
            pl.BlockSpec((BN, DN), lambda i: (i, 0)),
        ],
        out_specs=pl.BlockSpec((BN, DN), lambda i: (i, 0)),
        out_shape=jax.ShapeDtypeStruct((N, DN), jnp.float32),
    )(p, x)


def kernel(x, edge_index, edge_attr, W_node1, b_node1, W_node2, W_edge1,
           b_edge1, W_edge2, W_combine, W_update_edge):
    ei = edge_index.astype(jnp.int32)
    src3 = ei[0].reshape(NW, NCHUNK, CH)
    dst3 = ei[1].reshape(NW, NCHUNK, CH)
    hs, hd = _sc_gather(x, src3, dst3)
    m = _tc_edge(hs, hd, edge_attr,
                 W_node1, b_node1.reshape(1, DN), W_node2,
                 W_edge1, b_edge1.reshape(1, DN), W_edge2,
                 W_combine, W_update_edge)
    p = _sc_scatter(m, dst3, jnp.zeros((N_PAD, DN), jnp.float32))
    return _tc_combine(p, x)


# speedup vs baseline: 4.3703x; 1.0680x over previous
import jax
import jax.numpy as jnp
from jax import lax
from jax.experimental import pallas as pl
from jax.experimental.pallas import tpu as pltpu
from jax.experimental.pallas import tpu_sc as plsc

N = 10000
E = 320000
DN = 128
DE = 16
DP = DN // 2

NC, NS = 2, 16
NW = NC * NS
CH = 80
EPW = E // NW
NCHUNK = EPW // CH
N_PAD = 10240
TR = N_PAD // NS


def _gather_body(x_hbm, src_hbm, dst_hbm, hs_hbm, hd_hbm,
                 idx_s, idx_d, rs0, rs1, rd0, rd1,
                 sgs0, sgs1, sgd0, sgd1, sss0, sss1, ssd0, ssd1):
    c = lax.axis_index("c")
    s = lax.axis_index("s")
    w = c * NS + s
    base = w * EPW
    pltpu.sync_copy(src_hbm.at[w], idx_s)
    pltpu.sync_copy(dst_hbm.at[w], idx_d)

    rs = (rs0, rs1)
    rd = (rd0, rd1)
    sgs = (sgs0, sgs1)
    sgd = (sgd0, sgd1)
    sss = (sss0, sss1)
    ssd = (ssd0, ssd1)

    def issue_g(i, b):
        pltpu.async_copy(x_hbm.at[idx_s.at[i]], rs[b], sgs[b])
        pltpu.async_copy(x_hbm.at[idx_d.at[i]], rd[b], sgd[b])

    def wait_g(b):
        pltpu.make_async_copy(x_hbm.at[idx_s.at[0]], rs[b], sgs[b]).wait()
        pltpu.make_async_copy(x_hbm.at[idx_d.at[0]], rd[b], sgd[b]).wait()

    def issue_s(i, b):
        off = base + i * CH
        pltpu.async_copy(rs[b], hs_hbm.at[pl.ds(off, CH)], sss[b])
        pltpu.async_copy(rd[b], hd_hbm.at[pl.ds(off, CH)], ssd[b])

    def wait_s(b):
        pltpu.make_async_copy(rs[b], hs_hbm.at[pl.ds(base, CH)], sss[b]).wait()
        pltpu.make_async_copy(rd[b], hd_hbm.at[pl.ds(base, CH)], ssd[b]).wait()

    issue_g(0, 0)
    wait_g(0)
    issue_g(1, 1)
    issue_s(0, 0)
    wait_g(1)
    wait_s(0)
    issue_g(2, 0)
    issue_s(1, 1)

    def body(j, carry):
        for b in (0, 1):
            i = 2 * j + b
            wait_g(b)
            wait_s(1 - b)
            issue_g(i + 1, 1 - b)
            issue_s(i, b)
        return carry

    lax.fori_loop(1, (NCHUNK - 1) // 2, body, 0)
    wait_g(0)
    wait_s(1)
    issue_s(NCHUNK - 1, 0)
    wait_s(0)


def _sc_gather(x, src3, dst3):
    mesh = plsc.VectorSubcoreMesh(core_axis_name="c", subcore_axis_name="s")
    f = pl.kernel(
        _gather_body,
        out_type=(
            jax.ShapeDtypeStruct((E, DN), jnp.float32),
            jax.ShapeDtypeStruct((E, DN), jnp.float32),
        ),
        mesh=mesh,
        scratch_types=[
            pltpu.VMEM((NCHUNK, CH), jnp.int32),
            pltpu.VMEM((NCHUNK, CH), jnp.int32),
            pltpu.VMEM((CH, DN), jnp.float32),
            pltpu.VMEM((CH, DN), jnp.float32),
            pltpu.VMEM((CH, DN), jnp.float32),
            pltpu.VMEM((CH, DN), jnp.float32),
        ] + [pltpu.SemaphoreType.DMA] * 8,
    )
    return f(x, src3, dst3)


BE = 8000


def _edge_body(hs_ref, hd_ref, ea_ref, wn1_ref, bn1_ref, wn2_ref, we1_ref,
               be1_ref, we2_ref, wc_ref, wue_ref, m_ref):
    hs = hs_ref[...]
    hd = hd_ref[...]
    m1 = jnp.maximum(hs @ wn1_ref[...] + bn1_ref[...], 0.0) @ wn2_ref[...]
    u = (hs * hd) @ wue_ref[...]
    e_h = 0.8 * ea_ref[...] + 0.2 * u
    t = jnp.maximum(e_h @ we1_ref[...] + be1_ref[...], 0.0)
    m2 = t @ we2_ref[...]
    m_ref[...] = jnp.tanh((m1 * m2) @ wc_ref[...])


def _tc_edge(hs, hd, ea, wn1, bn1, wn2, we1, be1, we2, wc, wue):
    full = lambda shape: pl.BlockSpec(shape, lambda i: (0,) * len(shape))
    return pl.pallas_call(
        _edge_body,
        grid=(E // BE,),
        in_specs=[
            pl.BlockSpec((BE, DN), lambda i: (i, 0)),
            pl.BlockSpec((BE, DN), lambda i: (i, 0)),
            pl.BlockSpec((BE, DE), lambda i: (i, 0)),
            full((DN, DN)), full((1, DN)), full((DN, DN)),
            full((DE, DN)), full((1, DN)), full((DN, DN)),
            full((DN, DN)), full((DN, DE)),
        ],
        out_specs=pl.BlockSpec((BE, DN), lambda i: (i, 0)),
        out_shape=jax.ShapeDtypeStruct((E, DN), jnp.float32),
    )(hs, hd, ea, wn1, bn1, wn2, we1, be1, we2, wc, wue)


def _scatter_body(m_hbm, dst_hbm, zero_hbm, out_hbm, idx_all, idx0, idx1,
                  mb0, mb1, acc, sl0, sl1):
    c = lax.axis_index("c")
    s = lax.axis_index("s")
    w = c * NS + s
    base = w * EPW
    pltpu.sync_copy(zero_hbm.at[pl.ds(s * TR, TR)], acc.at[pl.ds(s * TR, TR)])
    pltpu.sync_copy(dst_hbm.at[w], idx_all)

    idx = (idx0, idx1)
    mb = (mb0, mb1)
    sl = (sl0, sl1)

    def issue_l(i, b):
        pltpu.async_copy(m_hbm.at[pl.ds(base + i * CH, CH)], mb[b], sl[b])

    def wait_l(b):
        pltpu.make_async_copy(m_hbm.at[pl.ds(base, CH)], mb[b], sl[b]).wait()

    def scatter_add(i, b):
        for k in range(CH // 16):
            idx[b][pl.ds(k * 16, 16)] = idx_all[i, pl.ds(k * 16, 16)]
        pltpu.sync_copy(mb[b], acc.at[idx[b]], add=True)

    issue_l(0, 0)
    issue_l(1, 1)
    plsc.subcore_barrier()

    def body(j, carry):
        for b in (0, 1):
            i = 2 * j + b
            wait_l(b)
            scatter_add(i, b)
            issue_l(i + 2, b)
        return carry

    lax.fori_loop(0, (NCHUNK - 3) // 2, body, 0)
    wait_l(0)
    scatter_add(NCHUNK - 3, 0)
    issue_l(NCHUNK - 1, 0)
    wait_l(1)
    scatter_add(NCHUNK - 2, 1)
    wait_l(0)
    scatter_add(NCHUNK - 1, 0)

    plsc.subcore_barrier()
    pltpu.sync_copy(acc.at[pl.ds(s * TR, TR)], out_hbm.at[c, pl.ds(s * TR, TR)])


def _sc_scatter(m, dst3, zeros_n):
    mesh = plsc.VectorSubcoreMesh(core_axis_name="c", subcore_axis_name="s")
    f = pl.kernel(
        _scatter_body,
        out_type=jax.ShapeDtypeStruct((NC, N_PAD, DN), jnp.float32),
        mesh=mesh,
        scratch_types=[
            pltpu.VMEM((NCHUNK, CH), jnp.int32),
            pltpu.VMEM((CH,), jnp.int32),
            pltpu.VMEM((CH,), jnp.int32),
            pltpu.VMEM((CH, DN), jnp.float32),
            pltpu.VMEM((CH, DN), jnp.float32),
            pltpu.VMEM_SHARED((N_PAD, DN), jnp.float32),
        ] + [pltpu.SemaphoreType.DMA] * 2,
    )
    return f(m, dst3, zeros_n)


BN = 2000


def _combine_body(p_ref, x_ref, o_ref):
    o_ref[...] = p_ref[0] + p_ref[1] + x_ref[...]


def _tc_combine(p, x):
    return pl.pallas_call(
        _combine_body,
        grid=(N // BN,),
        in_specs=[
            pl.BlockSpec((NC, BN, DN), lambda i: (0, i, 0)),
            pl.BlockSpec((BN, DN), lambda i: (i, 0)),
        ],
        out_specs=pl.BlockSpec((BN, DN), lambda i: (i, 0)),
        out_shape=jax.ShapeDtypeStruct((N, DN), jnp.float32),
    )(p, x)


def kernel(x, edge_index, edge_attr, W_node1, b_node1, W_node2, W_edge1,
           b_edge1, W_edge2, W_combine, W_update_edge):
    ei = edge_index.astype(jnp.int32)
    src3 = ei[0].reshape(NW, NCHUNK, CH)
    dst3 = ei[1].reshape(NW, NCHUNK, CH)
    hs, hd = _sc_gather(x, src3, dst3)
    m = _tc_edge(hs, hd, edge_attr,
                 W_node1, b_node1.reshape(1, DN), W_node2,
                 W_edge1, b_edge1.reshape(1, DN), W_edge2,
                 W_combine, W_update_edge)
    p = _sc_scatter(m, dst3, jnp.zeros((N_PAD, DN), jnp.float32))
    return _tc_combine(p, x)


# speedup vs baseline: 4.4112x; 1.0094x over previous
import jax
import jax.numpy as jnp
from jax import lax
from jax.experimental import pallas as pl
from jax.experimental.pallas import tpu as pltpu
from jax.experimental.pallas import tpu_sc as plsc

N = 10000
E = 320000
DN = 128
DE = 16
DP = DN // 2

NC, NS = 2, 16
NW = NC * NS
CH = 80
EPW = E // NW
NCHUNK = EPW // CH
N_PAD = 10240
TR = N_PAD // NS


def _gather_body(x_hbm, src_hbm, dst_hbm, hs_hbm, hd_hbm,
                 idx_s, idx_d,
                 rs0, rs1, rs2, rs3, rd0, rd1, rd2, rd3,
                 *sems):
    c = lax.axis_index("c")
    s = lax.axis_index("s")
    w = c * NS + s
    base = w * EPW
    pltpu.sync_copy(src_hbm.at[w], idx_s)
    pltpu.sync_copy(dst_hbm.at[w], idx_d)

    rs = (rs0, rs1, rs2, rs3)
    rd = (rd0, rd1, rd2, rd3)
    sgs = sems[0:4]
    sgd = sems[4:8]
    sss = sems[8:12]
    ssd = sems[12:16]

    def issue_g(i, b):
        pltpu.async_copy(x_hbm.at[idx_s.at[i]], rs[b], sgs[b])
        pltpu.async_copy(x_hbm.at[idx_d.at[i]], rd[b], sgd[b])

    def wait_g(b):
        pltpu.make_async_copy(x_hbm.at[idx_s.at[0]], rs[b], sgs[b]).wait()
        pltpu.make_async_copy(x_hbm.at[idx_d.at[0]], rd[b], sgd[b]).wait()

    def issue_s(i, b):
        off = base + i * CH
        pltpu.async_copy(rs[b], hs_hbm.at[pl.ds(off, CH)], sss[b])
        pltpu.async_copy(rd[b], hd_hbm.at[pl.ds(off, CH)], ssd[b])

    def wait_s(b):
        pltpu.make_async_copy(rs[b], hs_hbm.at[pl.ds(base, CH)], sss[b]).wait()
        pltpu.make_async_copy(rd[b], hd_hbm.at[pl.ds(base, CH)], ssd[b]).wait()

    issue_g(0, 0)
    issue_g(1, 1)
    issue_g(2, 2)
    wait_g(0)
    issue_s(0, 0)
    issue_g(3, 3)
    wait_g(1)
    issue_s(1, 1)

    def body(j, carry):
        for k in range(4):
            i = 2 + 4 * j + k
            b = (2 + k) % 4
            b2 = k % 4
            wait_s(b2)
            issue_g(i + 2, b2)
            wait_g(b)
            issue_s(i, b)
        return carry

    lax.fori_loop(0, (NCHUNK - 5) // 4, body, 0)
    wait_s(0)
    issue_g(NCHUNK - 1, 0)
    wait_g(2)
    issue_s(NCHUNK - 3, 2)
    wait_s(1)
    wait_g(3)
    issue_s(NCHUNK - 2, 3)
    wait_s(2)
    wait_g(0)
    issue_s(NCHUNK - 1, 0)
    wait_s(3)
    wait_s(0)


def _sc_gather(x, src3, dst3):
    mesh = plsc.VectorSubcoreMesh(core_axis_name="c", subcore_axis_name="s")
    f = pl.kernel(
        _gather_body,
        out_type=(
            jax.ShapeDtypeStruct((E, DN), jnp.float32),
            jax.ShapeDtypeStruct((E, DN), jnp.float32),
        ),
        mesh=mesh,
        scratch_types=[
            pltpu.VMEM((NCHUNK, CH), jnp.int32),
            pltpu.VMEM((NCHUNK, CH), jnp.int32),
        ] + [pltpu.VMEM((CH, DN), jnp.float32)] * 8
          + [pltpu.SemaphoreType.DMA] * 16,
    )
    return f(x, src3, dst3)


BE = 8000


def _edge_body(hs_ref, hd_ref, ea_ref, wn1_ref, bn1_ref, wn2_ref, we1_ref,
               be1_ref, we2_ref, wc_ref, wue_ref, m_ref):
    hs = hs_ref[...]
    hd = hd_ref[...]
    m1 = jnp.maximum(hs @ wn1_ref[...] + bn1_ref[...], 0.0) @ wn2_ref[...]
    u = (hs * hd) @ wue_ref[...]
    e_h = 0.8 * ea_ref[...] + 0.2 * u
    t = jnp.maximum(e_h @ we1_ref[...] + be1_ref[...], 0.0)
    m2 = t @ we2_ref[...]
    m_ref[...] = jnp.tanh((m1 * m2) @ wc_ref[...])


def _tc_edge(hs, hd, ea, wn1, bn1, wn2, we1, be1, we2, wc, wue):
    full = lambda shape: pl.BlockSpec(shape, lambda i: (0,) * len(shape))
    return pl.pallas_call(
        _edge_body,
        grid=(E // BE,),
        in_specs=[
            pl.BlockSpec((BE, DN), lambda i: (i, 0)),
            pl.BlockSpec((BE, DN), lambda i: (i, 0)),
            pl.BlockSpec((BE, DE), lambda i: (i, 0)),
            full((DN, DN)), full((1, DN)), full((DN, DN)),
            full((DE, DN)), full((1, DN)), full((DN, DN)),
            full((DN, DN)), full((DN, DE)),
        ],
        out_specs=pl.BlockSpec((BE, DN), lambda i: (i, 0)),
        out_shape=jax.ShapeDtypeStruct((E, DN), jnp.float32),
    )(hs, hd, ea, wn1, bn1, wn2, we1, be1, we2, wc, wue)


def _scatter_body(m_hbm, dst_hbm, zero_hbm, out_hbm, idx_all, idx0, idx1,
                  mb0, mb1, acc, sl0, sl1):
    c = lax.axis_index("c")
    s = lax.axis_index("s")
    w = c * NS + s
    base = w * EPW
    pltpu.sync_copy(zero_hbm.at[pl.ds(s * TR, TR)], acc.at[pl.ds(s * TR, TR)])
    pltpu.sync_copy(dst_hbm.at[w], idx_all)

    idx = (idx0, idx1)
    mb = (mb0, mb1)
    sl = (sl0, sl1)

    def issue_l(i, b):
        pltpu.async_copy(m_hbm.at[pl.ds(base + i * CH, CH)], mb[b], sl[b])

    def wait_l(b):
        pltpu.make_async_copy(m_hbm.at[pl.ds(base, CH)], mb[b], sl[b]).wait()

    def scatter_add(i, b):
        for k in range(CH // 16):
            idx[b][pl.ds(k * 16, 16)] = idx_all[i, pl.ds(k * 16, 16)]
        pltpu.sync_copy(mb[b], acc.at[idx[b]], add=True)

    issue_l(0, 0)
    issue_l(1, 1)
    plsc.subcore_barrier()

    def body(j, carry):
        for b in (0, 1):
            i = 2 * j + b
            wait_l(b)
            scatter_add(i, b)
            issue_l(i + 2, b)
        return carry

    lax.fori_loop(0, (NCHUNK - 3) // 2, body, 0)
    wait_l(0)
    scatter_add(NCHUNK - 3, 0)
    issue_l(NCHUNK - 1, 0)
    wait_l(1)
    scatter_add(NCHUNK - 2, 1)
    wait_l(0)
    scatter_add(NCHUNK - 1, 0)

    plsc.subcore_barrier()
    pltpu.sync_copy(acc.at[pl.ds(s * TR, TR)], out_hbm.at[c, pl.ds(s * TR, TR)])


def _sc_scatter(m, dst3, zeros_n):
    mesh = plsc.VectorSubcoreMesh(core_axis_name="c", subcore_axis_name="s")
    f = pl.kernel(
        _scatter_body,
        out_type=jax.ShapeDtypeStruct((NC, N_PAD, DN), jnp.float32),
        mesh=mesh,
        scratch_types=[
            pltpu.VMEM((NCHUNK, CH), jnp.int32),
            pltpu.VMEM((CH,), jnp.int32),
            pltpu.VMEM((CH,), jnp.int32),
            pltpu.VMEM((CH, DN), jnp.float32),
            pltpu.VMEM((CH, DN), jnp.float32),
            pltpu.VMEM_SHARED((N_PAD, DN), jnp.float32),
        ] + [pltpu.SemaphoreType.DMA] * 2,
    )
    return f(m, dst3, zeros_n)


BN = 2000


def _combine_body(p_ref, x_ref, o_ref):
    o_ref[...] = p_ref[0] + p_ref[1] + x_ref[...]


def _tc_combine(p, x):
    return pl.pallas_call(
        _combine_body,
        grid=(N // BN,),
        in_specs=[
            pl.BlockSpec((NC, BN, DN), lambda i: (0, i, 0)),
            pl.BlockSpec((BN, DN), lambda i: (i, 0)),
        ],
        out_specs=pl.BlockSpec((BN, DN), lambda i: (i, 0)),
        out_shape=jax.ShapeDtypeStruct((N, DN), jnp.float32),
    )(p, x)


def kernel(x, edge_index, edge_attr, W_node1, b_node1, W_node2, W_edge1,
           b_edge1, W_edge2, W_combine, W_update_edge):
    ei = edge_index.astype(jnp.int32)
    src3 = ei[0].reshape(NW, NCHUNK, CH)
    dst3 = ei[1].reshape(NW, NCHUNK, CH)
    hs, hd = _sc_gather(x, src3, dst3)
    m = _tc_edge(hs, hd, edge_attr,
                 W_node1, b_node1.reshape(1, DN), W_node2,
                 W_edge1, b_edge1.reshape(1, DN), W_edge2,
                 W_combine, W_update_edge)
    p = _sc_scatter(m, dst3, jnp.zeros((N_PAD, DN), jnp.float32))
    return _tc_combine(p, x)


# speedup vs baseline: 4.4121x; 1.0002x over previous
import jax
import jax.numpy as jnp
from jax import lax
from jax.experimental import pallas as pl
from jax.experimental.pallas import tpu as pltpu
from jax.experimental.pallas import tpu_sc as plsc

N = 10000
E = 320000
DN = 128
DE = 16
DP = DN // 2

NC, NS = 2, 16
NW = NC * NS
CH = 80
EPW = E // NW
NCHUNK = EPW // CH
N_PAD = 10240
TR = N_PAD // NS


def _gather_body(x_hbm, src_hbm, dst_hbm, hs_hbm, hd_hbm,
                 idx_s, idx_d,
                 rs0, rs1, rs2, rs3, rd0, rd1, rd2, rd3,
                 *sems):
    c = lax.axis_index("c")
    s = lax.axis_index("s")
    w = c * NS + s
    base = w * EPW
    pltpu.sync_copy(src_hbm.at[w], idx_s)
    pltpu.sync_copy(dst_hbm.at[w], idx_d)

    rs = (rs0, rs1, rs2, rs3)
    rd = (rd0, rd1, rd2, rd3)
    sgs = sems[0:4]
    sgd = sems[4:8]
    sss = sems[8:12]
    ssd = sems[12:16]

    def issue_g(i, b):
        pltpu.async_copy(x_hbm.at[idx_s.at[i]], rs[b], sgs[b])
        pltpu.async_copy(x_hbm.at[idx_d.at[i]], rd[b], sgd[b])

    def wait_g(b):
        pltpu.make_async_copy(x_hbm.at[idx_s.at[0]], rs[b], sgs[b]).wait()
        pltpu.make_async_copy(x_hbm.at[idx_d.at[0]], rd[b], sgd[b]).wait()

    def issue_s(i, b):
        off = base + i * CH
        pltpu.async_copy(rs[b], hs_hbm.at[pl.ds(off, CH)], sss[b])
        pltpu.async_copy(rd[b], hd_hbm.at[pl.ds(off, CH)], ssd[b])

    def wait_s(b):
        pltpu.make_async_copy(rs[b], hs_hbm.at[pl.ds(base, CH)], sss[b]).wait()
        pltpu.make_async_copy(rd[b], hd_hbm.at[pl.ds(base, CH)], ssd[b]).wait()

    issue_g(0, 0)
    issue_g(1, 1)
    issue_g(2, 2)
    wait_g(0)
    issue_s(0, 0)
    issue_g(3, 3)
    wait_g(1)
    issue_s(1, 1)

    def body(j, carry):
        for k in range(4):
            i = 2 + 4 * j + k
            b = (2 + k) % 4
            b2 = k % 4
            wait_s(b2)
            issue_g(i + 2, b2)
            wait_g(b)
            issue_s(i, b)
        return carry

    lax.fori_loop(0, (NCHUNK - 5) // 4, body, 0)
    wait_s(0)
    issue_g(NCHUNK - 1, 0)
    wait_g(2)
    issue_s(NCHUNK - 3, 2)
    wait_s(1)
    wait_g(3)
    issue_s(NCHUNK - 2, 3)
    wait_s(2)
    wait_g(0)
    issue_s(NCHUNK - 1, 0)
    wait_s(3)
    wait_s(0)


def _sc_gather(x, src3, dst3):
    mesh = plsc.VectorSubcoreMesh(core_axis_name="c", subcore_axis_name="s")
    f = pl.kernel(
        _gather_body,
        out_type=(
            jax.ShapeDtypeStruct((E, DN), jnp.float32),
            jax.ShapeDtypeStruct((E, DN), jnp.float32),
        ),
        mesh=mesh,
        scratch_types=[
            pltpu.VMEM((NCHUNK, CH), jnp.int32),
            pltpu.VMEM((NCHUNK, CH), jnp.int32),
        ] + [pltpu.VMEM((CH, DN), jnp.float32)] * 8
          + [pltpu.SemaphoreType.DMA] * 16,
    )
    return f(x, src3, dst3)


BE = 10000


def _edge_body(hs_ref, hd_ref, ea_ref, wn1_ref, bn1_ref, wn2_ref, we1_ref,
               be1_ref, we2_ref, wc_ref, wue_ref, m_ref):
    hs = hs_ref[...]
    hd = hd_ref[...]
    m1 = jnp.maximum(hs @ wn1_ref[...] + bn1_ref[...], 0.0) @ wn2_ref[...]
    u = (hs * hd) @ wue_ref[...]
    e_h = 0.8 * ea_ref[...] + 0.2 * u
    t = jnp.maximum(e_h @ we1_ref[...] + be1_ref[...], 0.0)
    m2 = t @ we2_ref[...]
    m_ref[...] = jnp.tanh((m1 * m2) @ wc_ref[...])


def _tc_edge(hs, hd, ea, wn1, bn1, wn2, we1, be1, we2, wc, wue):
    full = lambda shape: pl.BlockSpec(shape, lambda i: (0,) * len(shape))
    return pl.pallas_call(
        _edge_body,
        grid=(E // BE,),
        in_specs=[
            pl.BlockSpec((BE, DN), lambda i: (i, 0)),
            pl.BlockSpec((BE, DN), lambda i: (i, 0)),
            pl.BlockSpec((BE, DE), lambda i: (i, 0)),
            full((DN, DN)), full((1, DN)), full((DN, DN)),
            full((DE, DN)), full((1, DN)), full((DN, DN)),
            full((DN, DN)), full((DN, DE)),
        ],
        out_specs=pl.BlockSpec((BE, DN), lambda i: (i, 0)),
        out_shape=jax.ShapeDtypeStruct((E, DN), jnp.float32),
    )(hs, hd, ea, wn1, bn1, wn2, we1, be1, we2, wc, wue)


def _scatter_body(m_hbm, dst_hbm, zero_hbm, out_hbm, idx_all, idx0, idx1,
                  mb0, mb1, acc, sl0, sl1):
    c = lax.axis_index("c")
    s = lax.axis_index("s")
    w = c * NS + s
    base = w * EPW
    pltpu.sync_copy(zero_hbm.at[pl.ds(s * TR, TR)], acc.at[pl.ds(s * TR, TR)])
    pltpu.sync_copy(dst_hbm.at[w], idx_all)

    idx = (idx0, idx1)
    mb = (mb0, mb1)
    sl = (sl0, sl1)

    def issue_l(i, b):
        pltpu.async_copy(m_hbm.at[pl.ds(base + i * CH, CH)], mb[b], sl[b])

    def wait_l(b):
        pltpu.make_async_copy(m_hbm.at[pl.ds(base, CH)], mb[b], sl[b]).wait()

    def scatter_add(i, b):
        for k in range(CH // 16):
            idx[b][pl.ds(k * 16, 16)] = idx_all[i, pl.ds(k * 16, 16)]
        pltpu.sync_copy(mb[b], acc.at[idx[b]], add=True)

    issue_l(0, 0)
    issue_l(1, 1)
    plsc.subcore_barrier()

    def body(j, carry):
        for b in (0, 1):
            i = 2 * j + b
            wait_l(b)
            scatter_add(i, b)
            issue_l(i + 2, b)
        return carry

    lax.fori_loop(0, (NCHUNK - 3) // 2, body, 0)
    wait_l(0)
    scatter_add(NCHUNK - 3, 0)
    issue_l(NCHUNK - 1, 0)
    wait_l(1)
    scatter_add(NCHUNK - 2, 1)
    wait_l(0)
    scatter_add(NCHUNK - 1, 0)

    plsc.subcore_barrier()
    pltpu.sync_copy(acc.at[pl.ds(s * TR, TR)], out_hbm.at[c, pl.ds(s * TR, TR)])


def _sc_scatter(m, dst3, zeros_n):
    mesh = plsc.VectorSubcoreMesh(core_axis_name="c", subcore_axis_name="s")
    f = pl.kernel(
        _scatter_body,
        out_type=jax.ShapeDtypeStruct((NC, N_PAD, DN), jnp.float32),
        mesh=mesh,
        scratch_types=[
            pltpu.VMEM((NCHUNK, CH), jnp.int32),
            pltpu.VMEM((CH,), jnp.int32),
            pltpu.VMEM((CH,), jnp.int32),
            pltpu.VMEM((CH, DN), jnp.float32),
            pltpu.VMEM((CH, DN), jnp.float32),
            pltpu.VMEM_SHARED((N_PAD, DN), jnp.float32),
        ] + [pltpu.SemaphoreType.DMA] * 2,
    )
    return f(m, dst3, zeros_n)


BN = 2000


def _combine_body(p_ref, x_ref, o_ref):
    o_ref[...] = p_ref[0] + p_ref[1] + x_ref[...]


def _tc_combine(p, x):
    return pl.pallas_call(
        _combine_body,
        grid=(N // BN,),
        in_specs=[
            pl.BlockSpec((NC, BN, DN), lambda i: (0, i, 0)),
            pl.BlockSpec((BN, DN), lambda i: (i, 0)),
        ],
        out_specs=pl.BlockSpec((BN, DN), lambda i: (i, 0)),
        out_shape=jax.ShapeDtypeStruct((N, DN), jnp.float32),
    )(p, x)


def kernel(x, edge_index, edge_attr, W_node1, b_node1, W_node2, W_edge1,
           b_edge1, W_edge2, W_combine, W_update_edge):
    ei = edge_index.astype(jnp.int32)
    src3 = ei[0].reshape(NW, NCHUNK, CH)
    dst3 = ei[1].reshape(NW, NCHUNK, CH)
    hs, hd = _sc_gather(x, src3, dst3)
    m = _tc_edge(hs, hd, edge_attr,
                 W_node1, b_node1.reshape(1, DN), W_node2,
                 W_edge1, b_edge1.reshape(1, DN), W_edge2,
                 W_combine, W_update_edge)
    p = _sc_scatter(m, dst3, jnp.zeros((N_PAD, DN), jnp.float32))
    return _tc_combine(p, x)
